# Initial kernel scaffold; baseline (speedup 1.0000x reference)
import jax
import jax.numpy as jnp
from jax import lax
from jax.experimental import pallas as pl
from jax.experimental.pallas import tpu as pltpu
from jax.experimental.pallas import tpu_sc as plsc

N = 10000
NNZ = 160000
IN_DIM, HIDDEN_DIM, OUT_DIM = 256, 512, 256
NC = 2
NS = 16
L = 16
HALF = IN_DIM // 2
K = 80
EPT = NNZ // NS
NB = EPT // K
RPT = N // NS
RB = 25
NRB = RPT // RB

_SC_MESH = plsc.VectorSubcoreMesh(
    core_axis_name="c", subcore_axis_name="s", num_cores=NC, num_subcores=NS
)


def _counts_body(nidx_hbm, hidx_hbm, dinv_hbm, binv_hbm,
                 idx_v, ones_v, zbuf, cbuf, accum):
    cid = lax.axis_index("c")
    sid = lax.axis_index("s")
    row0 = sid * RPT

    for r in range(K):
        ones_v[r, :] = jnp.ones((L,), jnp.float32)
    for r in range(RB):
        zbuf[r, :] = jnp.zeros((L,), jnp.float32)

    def _zero(b, carry):
        pltpu.sync_copy(zbuf, accum.at[pl.ds(row0 + b * RB, RB)])
        return carry

    lax.fori_loop(0, NRB, _zero, 0)
    plsc.subcore_barrier()

    def _run(idx_hbm, out_hbm):
        pltpu.sync_copy(idx_hbm.at[sid], idx_v)

        def _batch(j, carry):
            pltpu.sync_copy(ones_v, accum.at[idx_v.at[j]], add=True)
            return carry

        lax.fori_loop(0, NB, _batch, 0)
        plsc.subcore_barrier()

        def _drain(b, carry):
            pltpu.sync_copy(accum.at[pl.ds(row0 + b * RB, RB)], cbuf)
            for r in range(RB):
                cnt = cbuf[r, :]
                cbuf[r, :] = jnp.where(cnt > 0.0, 1.0 / cnt, 0.0)
            pltpu.sync_copy(cbuf, out_hbm.at[pl.ds(row0 + b * RB, RB)])
            return carry

        lax.fori_loop(0, NRB, _drain, 0)

    @pl.when(cid == 0)
    def _():
        _run(nidx_hbm, dinv_hbm)

    @pl.when(cid == 1)
    def _():
        _run(hidx_hbm, binv_hbm)


_counts_call = pl.kernel(
    _counts_body,
    out_type=(
        jax.ShapeDtypeStruct((N, L), jnp.float32),
        jax.ShapeDtypeStruct((N, L), jnp.float32),
    ),
    mesh=_SC_MESH,
    scratch_types=(
        pltpu.VMEM((NB, K), jnp.int32),
        pltpu.VMEM((K, L), jnp.float32),
        pltpu.VMEM((RB, L), jnp.float32),
        pltpu.VMEM((RB, L), jnp.float32),
        pltpu.VMEM_SHARED((N, L), jnp.float32),
    ),
)


def _hop_body(gidx_hbm, sidx_hbm, ta_hbm, tb_hbm, scale_hbm, ba_hbm, bb_hbm,
              oa_hbm, ob_hbm,
              gidx_v, sidx_v, rows_v, zbuf, dbuf, scale_v, bias_v, accum):
    cid = lax.axis_index("c")
    sid = lax.axis_index("s")
    row0 = sid * RPT

    pltpu.sync_copy(gidx_hbm.at[sid], gidx_v)
    pltpu.sync_copy(sidx_hbm.at[sid], sidx_v)
    pltpu.sync_copy(scale_hbm.at[pl.ds(row0, RPT)], scale_v)

    for r in range(RB):
        for c in range(HALF // L):
            zbuf[r, pl.ds(c * L, L)] = jnp.zeros((L,), jnp.float32)

    def _zero(b, carry):
        pltpu.sync_copy(zbuf, accum.at[pl.ds(row0 + b * RB, RB)])
        return carry

    lax.fori_loop(0, NRB, _zero, 0)
    plsc.subcore_barrier()

    def _run(table_hbm, bias_ref, out_hbm):
        pltpu.sync_copy(bias_ref, bias_v)

        def _batch(j, carry):
            pltpu.sync_copy(table_hbm.at[gidx_v.at[j]], rows_v)
            pltpu.sync_copy(rows_v, accum.at[sidx_v.at[j]], add=True)
            return carry

        lax.fori_loop(0, NB, _batch, 0)
        plsc.subcore_barrier()

        bias_c = [bias_v[pl.ds(c * L, L)] for c in range(HALF // L)]

        def _drain(b, carry):
            pltpu.sync_copy(accum.at[pl.ds(row0 + b * RB, RB)], dbuf)
            for r in range(RB):
                sc = scale_v[b * RB + r, :]
                for c in range(HALF // L):
                    dbuf[r, pl.ds(c * L, L)] = (
                        dbuf[r, pl.ds(c * L, L)] * sc + bias_c[c]
                    )
            pltpu.sync_copy(dbuf, out_hbm.at[pl.ds(row0 + b * RB, RB)])
            return carry

        lax.fori_loop(0, NRB, _drain, 0)

    @pl.when(cid == 0)
    def _():
        _run(ta_hbm, ba_hbm, oa_hbm)

    @pl.when(cid == 1)
    def _():
        _run(tb_hbm, bb_hbm, ob_hbm)


_hop_call = pl.kernel(
    _hop_body,
    out_type=(
        jax.ShapeDtypeStruct((N, HALF), jnp.float32),
        jax.ShapeDtypeStruct((N, HALF), jnp.float32),
    ),
    mesh=_SC_MESH,
    scratch_types=(
        pltpu.VMEM((NB, K), jnp.int32),
        pltpu.VMEM((NB, K), jnp.int32),
        pltpu.VMEM((K, HALF), jnp.float32),
        pltpu.VMEM((RB, HALF), jnp.float32),
        pltpu.VMEM((RB, HALF), jnp.float32),
        pltpu.VMEM((RPT, L), jnp.float32),
        pltpu.VMEM((HALF,), jnp.float32),
        pltpu.VMEM_SHARED((N, HALF), jnp.float32),
    ),
)


BM = 1000


def _mm_body(aa_ref, ab_ref, w1_ref, b1_ref, w2_ref, ga_ref, gb_ref):
    a = jnp.concatenate([aa_ref[...], ab_ref[...]], axis=1)
    h = jnp.dot(a, w1_ref[...], preferred_element_type=jnp.float32)
    h = jnp.maximum(h + b1_ref[...], 0.0)
    g = jnp.dot(h, w2_ref[...], preferred_element_type=jnp.float32)
    ga_ref[...] = g[:, :HALF]
    gb_ref[...] = g[:, HALF:]


def _matmul(a1a, a1b, W1, b1, W2):
    return pl.pallas_call(
        _mm_body,
        grid=(N // BM,),
        in_specs=[
            pl.BlockSpec((BM, HALF), lambda i: (i, 0)),
            pl.BlockSpec((BM, HALF), lambda i: (i, 0)),
            pl.BlockSpec((IN_DIM, HIDDEN_DIM), lambda i: (0, 0)),
            pl.BlockSpec((1, HIDDEN_DIM), lambda i: (0, 0)),
            pl.BlockSpec((HIDDEN_DIM, OUT_DIM), lambda i: (0, 0)),
        ],
        out_specs=[
            pl.BlockSpec((BM, HALF), lambda i: (i, 0)),
            pl.BlockSpec((BM, HALF), lambda i: (i, 0)),
        ],
        out_shape=[
            jax.ShapeDtypeStruct((N, HALF), jnp.float32),
            jax.ShapeDtypeStruct((N, HALF), jnp.float32),
        ],
    )(a1a, a1b, W1, b1.reshape(1, HIDDEN_DIM), W2)


def kernel(x, edge_index, W1, b1, W2, b2):
    nidx = edge_index[0].reshape(NS, NB, K)
    hidx = edge_index[1].reshape(NS, NB, K)
    xa = x[:, :HALF]
    xb = x[:, HALF:]
    zbias = jnp.zeros((HALF,), jnp.float32)

    dinv, binv = _counts_call(nidx, hidx)
    s1a, s1b = _hop_call(nidx, hidx, xa, xb, binv, zbias, zbias)
    a1a, a1b = _hop_call(hidx, nidx, s1a, s1b, dinv, zbias, zbias)
    ga, gb = _matmul(a1a, a1b, W1, b1, W2)
    s2a, s2b = _hop_call(nidx, hidx, ga, gb, binv, zbias, zbias)
    oa, ob = _hop_call(hidx, nidx, s2a, s2b, dinv, b2[:HALF], b2[HALF:])
    return jnp.concatenate([oa, ob], axis=1)


# speedup vs baseline: 8.1787x; 8.1787x over previous
import jax
import jax.numpy as jnp
from jax import lax
from jax.experimental import pallas as pl
from jax.experimental.pallas import tpu as pltpu
from jax.experimental.pallas import tpu_sc as plsc

N = 10000
NNZ = 160000
IN_DIM, HIDDEN_DIM, OUT_DIM = 256, 512, 256
NC = 2
NS = 16
L = 16
HALF = IN_DIM // 2
K = 80
EPT = NNZ // NS
NB = EPT // K
RPT = N // NS
RB = 25
NRB = RPT // RB

_SC_MESH = plsc.VectorSubcoreMesh(
    core_axis_name="c", subcore_axis_name="s", num_cores=NC, num_subcores=NS
)
_SC_PARAMS = pltpu.CompilerParams(use_tc_tiling_on_sc=False)


def _counts_body(nidx_hbm, hidx_hbm, dinv_hbm, binv_hbm,
                 idx_v, ones_v, zbuf, cbuf, accum):
    cid = lax.axis_index("c")
    sid = lax.axis_index("s")
    row0 = sid * RPT

    for r in range(K):
        ones_v[r, :] = jnp.ones((L,), jnp.float32)
    for r in range(RB):
        zbuf[r, :] = jnp.zeros((L,), jnp.float32)

    def _zero(b, carry):
        pltpu.sync_copy(zbuf, accum.at[pl.ds(row0 + b * RB, RB)])
        return carry

    lax.fori_loop(0, NRB, _zero, 0)
    plsc.subcore_barrier()

    def _run(idx_hbm, out_hbm):
        pltpu.sync_copy(idx_hbm.at[sid], idx_v)

        def _batch(j, carry):
            pltpu.sync_copy(ones_v, accum.at[idx_v.at[j]], add=True)
            return carry

        lax.fori_loop(0, NB, _batch, 0)
        plsc.subcore_barrier()

        def _drain(b, carry):
            pltpu.sync_copy(accum.at[pl.ds(row0 + b * RB, RB)], cbuf)
            for r in range(RB):
                cnt = cbuf[r, :]
                cbuf[r, :] = jnp.where(cnt > 0.0, 1.0 / cnt, 0.0)
            pltpu.sync_copy(cbuf, out_hbm.at[pl.ds(row0 + b * RB, RB)])
            return carry

        lax.fori_loop(0, NRB, _drain, 0)

    @pl.when(cid == 0)
    def _():
        _run(nidx_hbm, dinv_hbm)

    @pl.when(cid == 1)
    def _():
        _run(hidx_hbm, binv_hbm)


_counts_call = pl.kernel(
    _counts_body,
    out_type=(
        jax.ShapeDtypeStruct((N, L), jnp.float32),
        jax.ShapeDtypeStruct((N, L), jnp.float32),
    ),
    mesh=_SC_MESH,
    compiler_params=_SC_PARAMS,
    scratch_types=(
        pltpu.VMEM((NB, K), jnp.int32),
        pltpu.VMEM((K, L), jnp.float32),
        pltpu.VMEM((RB, L), jnp.float32),
        pltpu.VMEM((RB, L), jnp.float32),
        pltpu.VMEM_SHARED((N, L), jnp.float32),
    ),
)


def _hop_body(gidx_hbm, sidx_hbm, ta_hbm, tb_hbm, scale_hbm, ba_hbm, bb_hbm,
              oa_hbm, ob_hbm,
              gidx_v, sidx_v, rows_v, zbuf, dbuf, scale_v, bias_v, accum):
    cid = lax.axis_index("c")
    sid = lax.axis_index("s")
    row0 = sid * RPT

    pltpu.sync_copy(gidx_hbm.at[sid], gidx_v)
    pltpu.sync_copy(sidx_hbm.at[sid], sidx_v)
    pltpu.sync_copy(scale_hbm.at[pl.ds(row0, RPT)], scale_v)

    for r in range(RB):
        for c in range(HALF // L):
            zbuf[r, pl.ds(c * L, L)] = jnp.zeros((L,), jnp.float32)

    def _zero(b, carry):
        pltpu.sync_copy(zbuf, accum.at[pl.ds(row0 + b * RB, RB)])
        return carry

    lax.fori_loop(0, NRB, _zero, 0)
    plsc.subcore_barrier()

    def _run(table_hbm, bias_ref, out_hbm):
        pltpu.sync_copy(bias_ref, bias_v)

        def _batch(j, carry):
            pltpu.sync_copy(table_hbm.at[gidx_v.at[j]], rows_v)
            pltpu.sync_copy(rows_v, accum.at[sidx_v.at[j]], add=True)
            return carry

        lax.fori_loop(0, NB, _batch, 0)
        plsc.subcore_barrier()

        bias_c = [bias_v[pl.ds(c * L, L)] for c in range(HALF // L)]

        def _drain(b, carry):
            pltpu.sync_copy(accum.at[pl.ds(row0 + b * RB, RB)], dbuf)
            for r in range(RB):
                sc = scale_v[b * RB + r, :]
                for c in range(HALF // L):
                    dbuf[r, pl.ds(c * L, L)] = (
                        dbuf[r, pl.ds(c * L, L)] * sc + bias_c[c]
                    )
            pltpu.sync_copy(dbuf, out_hbm.at[pl.ds(row0 + b * RB, RB)])
            return carry

        lax.fori_loop(0, NRB, _drain, 0)

    @pl.when(cid == 0)
    def _():
        _run(ta_hbm, ba_hbm, oa_hbm)

    @pl.when(cid == 1)
    def _():
        _run(tb_hbm, bb_hbm, ob_hbm)


_hop_call = pl.kernel(
    _hop_body,
    out_type=(
        jax.ShapeDtypeStruct((N, HALF), jnp.float32),
        jax.ShapeDtypeStruct((N, HALF), jnp.float32),
    ),
    mesh=_SC_MESH,
    compiler_params=_SC_PARAMS,
    scratch_types=(
        pltpu.VMEM((NB, K), jnp.int32),
        pltpu.VMEM((NB, K), jnp.int32),
        pltpu.VMEM((K, HALF), jnp.float32),
        pltpu.VMEM((RB, HALF), jnp.float32),
        pltpu.VMEM((RB, HALF), jnp.float32),
        pltpu.VMEM((RPT, L), jnp.float32),
        pltpu.VMEM((HALF,), jnp.float32),
        pltpu.VMEM_SHARED((N, HALF), jnp.float32),
    ),
)


BM = 1000


def _mm_body(aa_ref, ab_ref, w1_ref, b1_ref, w2_ref, ga_ref, gb_ref):
    a = jnp.concatenate([aa_ref[...], ab_ref[...]], axis=1)
    h = jnp.dot(a, w1_ref[...], preferred_element_type=jnp.float32)
    h = jnp.maximum(h + b1_ref[...], 0.0)
    g = jnp.dot(h, w2_ref[...], preferred_element_type=jnp.float32)
    ga_ref[...] = g[:, :HALF]
    gb_ref[...] = g[:, HALF:]


def _matmul(a1a, a1b, W1, b1, W2):
    return pl.pallas_call(
        _mm_body,
        grid=(N // BM,),
        in_specs=[
            pl.BlockSpec((BM, HALF), lambda i: (i, 0)),
            pl.BlockSpec((BM, HALF), lambda i: (i, 0)),
            pl.BlockSpec((IN_DIM, HIDDEN_DIM), lambda i: (0, 0)),
            pl.BlockSpec((1, HIDDEN_DIM), lambda i: (0, 0)),
            pl.BlockSpec((HIDDEN_DIM, OUT_DIM), lambda i: (0, 0)),
        ],
        out_specs=[
            pl.BlockSpec((BM, HALF), lambda i: (i, 0)),
            pl.BlockSpec((BM, HALF), lambda i: (i, 0)),
        ],
        out_shape=[
            jax.ShapeDtypeStruct((N, HALF), jnp.float32),
            jax.ShapeDtypeStruct((N, HALF), jnp.float32),
        ],
    )(a1a, a1b, W1, b1.reshape(1, HIDDEN_DIM), W2)


def kernel(x, edge_index, W1, b1, W2, b2):
    nidx = edge_index[0].reshape(NS, NB, K)
    hidx = edge_index[1].reshape(NS, NB, K)
    xa = x[:, :HALF]
    xb = x[:, HALF:]
    zbias = jnp.zeros((HALF,), jnp.float32)

    dinv, binv = _counts_call(nidx, hidx)
    s1a, s1b = _hop_call(nidx, hidx, xa, xb, binv, zbias, zbias)
    a1a, a1b = _hop_call(hidx, nidx, s1a, s1b, dinv, zbias, zbias)
    ga, gb = _matmul(a1a, a1b, W1, b1, W2)
    s2a, s2b = _hop_call(nidx, hidx, ga, gb, binv, zbias, zbias)
    oa, ob = _hop_call(hidx, nidx, s2a, s2b, dinv, b2[:HALF], b2[HALF:])
    return jnp.concatenate([oa, ob], axis=1)


# speedup vs baseline: 10.3119x; 1.2608x over previous
import jax
import jax.numpy as jnp
from jax import lax
from jax.experimental import pallas as pl
from jax.experimental.pallas import tpu as pltpu
from jax.experimental.pallas import tpu_sc as plsc

N = 10000
NNZ = 160000
IN_DIM, HIDDEN_DIM, OUT_DIM = 256, 512, 256
NC = 2
NS = 16
L = 16
HALF = IN_DIM // 2
K = 100
EPT = NNZ // NS
NB = EPT // K
RPT = N // NS
RB = 25
NRB = RPT // RB

_SC_MESH = plsc.VectorSubcoreMesh(
    core_axis_name="c", subcore_axis_name="s", num_cores=NC, num_subcores=NS
)
_SC_PARAMS = pltpu.CompilerParams(use_tc_tiling_on_sc=False)


def _counts_body(nidx_hbm, hidx_hbm, dinv_hbm, binv_hbm,
                 idx_v, ones_v, zbuf, cbuf, accum):
    cid = lax.axis_index("c")
    sid = lax.axis_index("s")
    row0 = sid * RPT

    for r in range(K):
        ones_v[r, :] = jnp.ones((L,), jnp.float32)
    for r in range(RB):
        zbuf[r, :] = jnp.zeros((L,), jnp.float32)

    def _zero(b, carry):
        pltpu.sync_copy(zbuf, accum.at[pl.ds(row0 + b * RB, RB)])
        return carry

    lax.fori_loop(0, NRB, _zero, 0)
    plsc.subcore_barrier()

    def _run(idx_hbm, out_hbm):
        pltpu.sync_copy(idx_hbm.at[sid], idx_v)

        def _batch(j, carry):
            pltpu.sync_copy(ones_v, accum.at[idx_v.at[j]], add=True)
            return carry

        lax.fori_loop(0, NB, _batch, 0)
        plsc.subcore_barrier()

        def _drain(b, carry):
            pltpu.sync_copy(accum.at[pl.ds(row0 + b * RB, RB)], cbuf)
            for r in range(RB):
                cnt = cbuf[r, :]
                cbuf[r, :] = jnp.where(cnt > 0.0, 1.0 / cnt, 0.0)
            pltpu.sync_copy(cbuf, out_hbm.at[pl.ds(row0 + b * RB, RB)])
            return carry

        lax.fori_loop(0, NRB, _drain, 0)

    @pl.when(cid == 0)
    def _():
        _run(nidx_hbm, dinv_hbm)

    @pl.when(cid == 1)
    def _():
        _run(hidx_hbm, binv_hbm)


_counts_call = pl.kernel(
    _counts_body,
    out_type=(
        jax.ShapeDtypeStruct((N, L), jnp.float32),
        jax.ShapeDtypeStruct((N, L), jnp.float32),
    ),
    mesh=_SC_MESH,
    compiler_params=_SC_PARAMS,
    scratch_types=(
        pltpu.VMEM((NB, K), jnp.int32),
        pltpu.VMEM((K, L), jnp.float32),
        pltpu.VMEM((RB, L), jnp.float32),
        pltpu.VMEM((RB, L), jnp.float32),
        pltpu.VMEM_SHARED((N, L), jnp.float32),
    ),
)


def _hop_body(gidx_hbm, sidx_hbm, ta_hbm, tb_hbm, scale_hbm, ba_hbm, bb_hbm,
              oa_hbm, ob_hbm,
              gidx_v, sidx_v, rows_v, rows_w, dbuf, sbuf, bias_v,
              accum, gs0, gs1, ss0, ss1):
    cid = lax.axis_index("c")
    sid = lax.axis_index("s")
    row0 = sid * RPT

    pltpu.sync_copy(gidx_hbm.at[sid], gidx_v)
    pltpu.sync_copy(sidx_hbm.at[sid], sidx_v)

    for r in range(RB):
        for c in range(HALF // L):
            dbuf[r, pl.ds(c * L, L)] = jnp.zeros((L,), jnp.float32)

    def _zero(b, carry):
        pltpu.sync_copy(dbuf, accum.at[pl.ds(row0 + b * RB, RB)])
        return carry

    lax.fori_loop(0, NRB, _zero, 0)
    plsc.subcore_barrier()

    def _run(table_hbm, bias_ref, out_hbm):
        pltpu.sync_copy(bias_ref, bias_v)

        def _gather(j, buf, sem):
            pltpu.async_copy(table_hbm.at[gidx_v.at[j]], buf, sem)

        def _gather_wait(j, buf, sem):
            pltpu.make_async_copy(table_hbm.at[gidx_v.at[j]], buf, sem).wait()

        def _scatter(j, buf, sem):
            pltpu.async_copy(buf, accum.at[sidx_v.at[j]], sem, add=True)

        def _scatter_wait(j, buf, sem):
            pltpu.make_async_copy(buf, accum.at[sidx_v.at[j]], sem).wait()

        _gather(0, rows_v, gs0)
        _gather(1, rows_w, gs1)

        def _pair(i, carry):
            j0 = 2 * i
            _gather_wait(j0, rows_v, gs0)
            _scatter(j0, rows_v, ss0)
            _gather_wait(j0 + 1, rows_w, gs1)
            _scatter(j0 + 1, rows_w, ss1)
            _scatter_wait(j0, rows_v, ss0)
            _gather(j0 + 2, rows_v, gs0)
            _scatter_wait(j0 + 1, rows_w, ss1)
            _gather(j0 + 3, rows_w, gs1)
            return carry

        lax.fori_loop(0, NB // 2 - 1, _pair, 0)
        jt = NB - 2
        _gather_wait(jt, rows_v, gs0)
        _scatter(jt, rows_v, ss0)
        _gather_wait(jt + 1, rows_w, gs1)
        _scatter(jt + 1, rows_w, ss1)
        _scatter_wait(jt, rows_v, ss0)
        _scatter_wait(jt + 1, rows_w, ss1)
        plsc.subcore_barrier()

        bias_c = [bias_v[pl.ds(c * L, L)] for c in range(HALF // L)]

        def _drain(b, carry):
            pltpu.sync_copy(accum.at[pl.ds(row0 + b * RB, RB)], dbuf)
            pltpu.sync_copy(scale_hbm.at[pl.ds(row0 + b * RB, RB)], sbuf)
            for r in range(RB):
                sc = sbuf[r, :]
                for c in range(HALF // L):
                    dbuf[r, pl.ds(c * L, L)] = (
                        dbuf[r, pl.ds(c * L, L)] * sc + bias_c[c]
                    )
            pltpu.sync_copy(dbuf, out_hbm.at[pl.ds(row0 + b * RB, RB)])
            return carry

        lax.fori_loop(0, NRB, _drain, 0)

    @pl.when(cid == 0)
    def _():
        _run(ta_hbm, ba_hbm, oa_hbm)

    @pl.when(cid == 1)
    def _():
        _run(tb_hbm, bb_hbm, ob_hbm)


_hop_call = pl.kernel(
    _hop_body,
    out_type=(
        jax.ShapeDtypeStruct((N, HALF), jnp.float32),
        jax.ShapeDtypeStruct((N, HALF), jnp.float32),
    ),
    mesh=_SC_MESH,
    compiler_params=_SC_PARAMS,
    scratch_types=(
        pltpu.VMEM((NB, K), jnp.int32),
        pltpu.VMEM((NB, K), jnp.int32),
        pltpu.VMEM((K, HALF), jnp.float32),
        pltpu.VMEM((K, HALF), jnp.float32),
        pltpu.VMEM((RB, HALF), jnp.float32),
        pltpu.VMEM((RB, L), jnp.float32),
        pltpu.VMEM((HALF,), jnp.float32),
        pltpu.VMEM_SHARED((N, HALF), jnp.float32),
        pltpu.SemaphoreType.DMA,
        pltpu.SemaphoreType.DMA,
        pltpu.SemaphoreType.DMA,
        pltpu.SemaphoreType.DMA,
    ),
)


BM = 1000


def _mm_body(aa_ref, ab_ref, w1_ref, b1_ref, w2_ref, ga_ref, gb_ref):
    a = jnp.concatenate([aa_ref[...], ab_ref[...]], axis=1)
    h = jnp.dot(a, w1_ref[...], preferred_element_type=jnp.float32)
    h = jnp.maximum(h + b1_ref[...], 0.0)
    g = jnp.dot(h, w2_ref[...], preferred_element_type=jnp.float32)
    ga_ref[...] = g[:, :HALF]
    gb_ref[...] = g[:, HALF:]


def _matmul(a1a, a1b, W1, b1, W2):
    return pl.pallas_call(
        _mm_body,
        grid=(N // BM,),
        in_specs=[
            pl.BlockSpec((BM, HALF), lambda i: (i, 0)),
            pl.BlockSpec((BM, HALF), lambda i: (i, 0)),
            pl.BlockSpec((IN_DIM, HIDDEN_DIM), lambda i: (0, 0)),
            pl.BlockSpec((1, HIDDEN_DIM), lambda i: (0, 0)),
            pl.BlockSpec((HIDDEN_DIM, OUT_DIM), lambda i: (0, 0)),
        ],
        out_specs=[
            pl.BlockSpec((BM, HALF), lambda i: (i, 0)),
            pl.BlockSpec((BM, HALF), lambda i: (i, 0)),
        ],
        out_shape=[
            jax.ShapeDtypeStruct((N, HALF), jnp.float32),
            jax.ShapeDtypeStruct((N, HALF), jnp.float32),
        ],
    )(a1a, a1b, W1, b1.reshape(1, HIDDEN_DIM), W2)


def kernel(x, edge_index, W1, b1, W2, b2):
    nidx = edge_index[0].reshape(NS, NB, K)
    hidx = edge_index[1].reshape(NS, NB, K)
    xa = x[:, :HALF]
    xb = x[:, HALF:]
    zbias = jnp.zeros((HALF,), jnp.float32)

    dinv, binv = _counts_call(nidx, hidx)
    s1a, s1b = _hop_call(nidx, hidx, xa, xb, binv, zbias, zbias)
    a1a, a1b = _hop_call(hidx, nidx, s1a, s1b, dinv, zbias, zbias)
    ga, gb = _matmul(a1a, a1b, W1, b1, W2)
    s2a, s2b = _hop_call(nidx, hidx, ga, gb, binv, zbias, zbias)
    oa, ob = _hop_call(hidx, nidx, s2a, s2b, dinv, b2[:HALF], b2[HALF:])
    return jnp.concatenate([oa, ob], axis=1)
